# baseline (speedup 1.0000x reference)
import functools

import jax
import jax.numpy as jnp
from jax import lax
from jax.experimental import pallas as pl
from jax.experimental.pallas import tpu as pltpu
from jax.experimental.pallas import tpu_sc as plsc

NUM_CORES = 2
NUM_SUBCORES = 16
NW = NUM_CORES * NUM_SUBCORES

B, L, C, D = 1024, 50, 26, 64
N = B * L
PER_W = N // NW
IDX_W = PER_W * C
SLOT_POS = 4
SLOT_IDX = SLOT_POS * C
NSLOT = PER_W // SLOT_POS
RING = 4
FLUSH_SLOTS = 40
FLUSH_POS = FLUSH_SLOTS * SLOT_POS
NFLUSH = NSLOT // FLUSH_SLOTS
CHUNKS = FLUSH_SLOTS // RING

LG = D // 16


def _body(table, idx, out, idx_v, rows_v, out_v, *sems):
    cid = lax.axis_index("c")
    sid = lax.axis_index("s")
    wid = sid * NUM_CORES + cid
    obase = wid * PER_W

    pltpu.sync_copy(idx.at[pl.ds(wid * NSLOT, NSLOT)], idx_v)

    def fire(s, b):
        pass

    def wait(s, b):
        pass

    for b in range(RING):
        fire(b, b)

    def flush_body(f, carry):
        slot0 = f * FLUSH_SLOTS

        def chunk_body(c2, carry2):
            base = slot0 + c2 * RING
            for b in range(RING):
                s = base + b
                wait(s, b)
                lp0 = (c2 * RING + b) * SLOT_POS

                for p in range(SLOT_POS):
                    r0 = p * C
                    for dg in range(LG):
                        out_v[lp0 + p, pl.ds(dg * 16, 16)] = rows_v[
                            b, r0, pl.ds(dg * 16, 16)
                        ]
                    for j in range(1, C):
                        for dg in range(LG):
                            plsc.addupdate(
                                out_v.at[lp0 + p, pl.ds(dg * 16, 16)],
                                rows_v[b, r0 + j, pl.ds(dg * 16, 16)],
                            )

                sn = s + RING

                @pl.when(sn < NSLOT)
                def _():
                    fire(sn, b)
            return carry2

        lax.fori_loop(0, CHUNKS, chunk_body, 0)
        pltpu.sync_copy(out_v, out.at[pl.ds(obase + f * FLUSH_POS, FLUSH_POS)])
        return carry

    lax.fori_loop(0, NFLUSH, flush_body, 0)


_embed_sum = functools.partial(
    pl.kernel,
    mesh=plsc.VectorSubcoreMesh(
        core_axis_name="c", subcore_axis_name="s",
        num_cores=NUM_CORES, num_subcores=NUM_SUBCORES,
    ),
    out_type=jax.ShapeDtypeStruct((N, D), jnp.float32),
    scratch_types=[
        pltpu.VMEM((NSLOT, SLOT_IDX), jnp.int32),
        pltpu.VMEM((RING, SLOT_IDX, D), jnp.float32),
        pltpu.VMEM((FLUSH_POS, D), jnp.float32),
    ]
    + [pltpu.SemaphoreType.DMA] * RING,
    compiler_params=pltpu.CompilerParams(use_tc_tiling_on_sc=False),
)(_body)


@jax.jit
def kernel(node_idx, clause_enc):
    idx2d = node_idx.astype(jnp.int32).reshape(NW * NSLOT, SLOT_IDX)
    out = _embed_sum(clause_enc, idx2d)
    return out.reshape(B, L, D)


# speedup vs baseline: 2.0162x; 2.0162x over previous
import functools

import jax
import jax.numpy as jnp
from jax import lax
from jax.experimental import pallas as pl
from jax.experimental.pallas import tpu as pltpu
from jax.experimental.pallas import tpu_sc as plsc

NUM_CORES = 2
NUM_SUBCORES = 16
NW = NUM_CORES * NUM_SUBCORES

B, L, C, D = 1024, 50, 26, 64
N = B * L
PER_W = N // NW
IDX_W = PER_W * C
SLOT_POS = 4
SLOT_IDX = SLOT_POS * C
NSLOT = PER_W // SLOT_POS
RING = 4
FLUSH_SLOTS = 40
FLUSH_POS = FLUSH_SLOTS * SLOT_POS
NFLUSH = NSLOT // FLUSH_SLOTS
CHUNKS = FLUSH_SLOTS // RING

LG = D // 16


def _body(table, idx, out, idx_v, rows_v, out_v, *sems):
    cid = lax.axis_index("c")
    sid = lax.axis_index("s")
    wid = sid * NUM_CORES + cid
    obase = wid * PER_W

    pltpu.sync_copy(idx.at[pl.ds(wid * NSLOT, NSLOT)], idx_v)

    def fire(s, b):
        pltpu.async_copy(table.at[idx_v.at[s]], rows_v.at[b], sems[b])

    def wait(s, b):
        pltpu.make_async_copy(table.at[idx_v.at[s]], rows_v.at[b], sems[b]).wait()

    for b in range(RING):
        fire(b, b)

    def flush_body(f, carry):
        slot0 = f * FLUSH_SLOTS

        def chunk_body(c2, carry2):
            base = slot0 + c2 * RING
            for b in range(RING):
                s = base + b
                wait(s, b)
                lp0 = (c2 * RING + b) * SLOT_POS

                for p in range(SLOT_POS):
                    r0 = p * C
                    accs = [
                        rows_v[b, r0, pl.ds(dg * 16, 16)] for dg in range(LG)
                    ]
                    for j in range(1, C):
                        for dg in range(LG):
                            accs[dg] = accs[dg] + rows_v[b, r0 + j, pl.ds(dg * 16, 16)]
                    for dg in range(LG):
                        out_v[lp0 + p, pl.ds(dg * 16, 16)] = accs[dg]

                sn = s + RING

                @pl.when(sn < NSLOT)
                def _():
                    fire(sn, b)
            return carry2

        lax.fori_loop(0, CHUNKS, chunk_body, 0)
        pltpu.sync_copy(out_v, out.at[pl.ds(obase + f * FLUSH_POS, FLUSH_POS)])
        return carry

    lax.fori_loop(0, NFLUSH, flush_body, 0)


_embed_sum = functools.partial(
    pl.kernel,
    mesh=plsc.VectorSubcoreMesh(
        core_axis_name="c", subcore_axis_name="s",
        num_cores=NUM_CORES, num_subcores=NUM_SUBCORES,
    ),
    out_type=jax.ShapeDtypeStruct((N, D), jnp.float32),
    scratch_types=[
        pltpu.VMEM((NSLOT, SLOT_IDX), jnp.int32),
        pltpu.VMEM((RING, SLOT_IDX, D), jnp.float32),
        pltpu.VMEM((FLUSH_POS, D), jnp.float32),
    ]
    + [pltpu.SemaphoreType.DMA] * RING,
    compiler_params=pltpu.CompilerParams(use_tc_tiling_on_sc=False),
)(_body)


@jax.jit
def kernel(node_idx, clause_enc):
    idx2d = node_idx.astype(jnp.int32).reshape(NW * NSLOT, SLOT_IDX)
    out = _embed_sum(clause_enc, idx2d)
    return out.reshape(B, L, D)


# speedup vs baseline: 3.3146x; 1.6439x over previous
import functools

import jax
import jax.numpy as jnp
from jax import lax
from jax.experimental import pallas as pl
from jax.experimental.pallas import tpu as pltpu
from jax.experimental.pallas import tpu_sc as plsc

NUM_CORES = 2
NUM_SUBCORES = 16
NW = NUM_CORES * NUM_SUBCORES

B, L, C, D = 1024, 50, 26, 64
N = B * L
PER_W = N // NW
P = 80
NBLK = PER_W // P
LG = D // 16


def _body(table, idx, out, idx_v, acc, *sems):
    sem_g = sems[0:2]
    sem_o = sems[2:4]

    cid = lax.axis_index("c")
    sid = lax.axis_index("s")
    wid = sid * NUM_CORES + cid
    obase = wid * PER_W

    pltpu.sync_copy(idx.at[:, pl.ds(wid * PER_W, PER_W)], idx_v)

    zero = jnp.zeros((16,), jnp.float32)

    def zero_buf(par):
        for r in range(P):
            for dg in range(LG):
                acc[par, r, pl.ds(dg * 16, 16)] = zero

    def fire_block(f, par):
        col0 = f * P
        for j in range(C):
            pltpu.async_copy(
                table.at[idx_v.at[j, pl.ds(col0, P)]],
                acc.at[par],
                sem_g[par],
                add=True,
            )

    def drain_block(f, par):
        col0 = f * P
        for j in range(C):
            pltpu.make_async_copy(
                table.at[idx_v.at[j, pl.ds(col0, P)]],
                acc.at[par],
                sem_g[par],
            ).wait()

    def fire_out(f, par):
        pltpu.async_copy(
            acc.at[par], out.at[pl.ds(obase + f * P, P)], sem_o[par]
        )

    def wait_out(f, par):
        pltpu.make_async_copy(
            acc.at[par], out.at[pl.ds(obase + f * P, P)], sem_o[par]
        ).wait()

    for par in range(2):
        zero_buf(par)
        fire_block(par, par)

    def body(i, carry):
        for par in range(2):
            f = 2 * i + par
            drain_block(f, par)
            fire_out(f, par)
            nf = f + 2

            @pl.when(nf < NBLK)
            def _():
                wait_out(f, par)
                zero_buf(par)
                fire_block(nf, par)

        return carry

    lax.fori_loop(0, NBLK // 2, body, 0)
    for par in range(2):
        wait_out(NBLK - 2 + par, par)


_embed_sum = functools.partial(
    pl.kernel,
    mesh=plsc.VectorSubcoreMesh(
        core_axis_name="c", subcore_axis_name="s",
        num_cores=NUM_CORES, num_subcores=NUM_SUBCORES,
    ),
    out_type=jax.ShapeDtypeStruct((N, D), jnp.float32),
    scratch_types=[
        pltpu.VMEM((C, PER_W), jnp.int32),
        pltpu.VMEM((2, P, D), jnp.float32),
    ]
    + [pltpu.SemaphoreType.DMA] * 4,
    compiler_params=pltpu.CompilerParams(use_tc_tiling_on_sc=False),
)(_body)


@jax.jit
def kernel(node_idx, clause_enc):
    idx_t = node_idx.astype(jnp.int32).reshape(N, C).T
    out = _embed_sum(clause_enc, idx_t)
    return out.reshape(B, L, D)


# speedup vs baseline: 3.3270x; 1.0038x over previous
import functools

import jax
import jax.numpy as jnp
from jax import lax
from jax.experimental import pallas as pl
from jax.experimental.pallas import tpu as pltpu
from jax.experimental.pallas import tpu_sc as plsc

NUM_CORES = 2
NUM_SUBCORES = 16
NW = NUM_CORES * NUM_SUBCORES

B, L, C, D = 1024, 50, 26, 64
N = B * L
PER_W = N // NW
P = 80
NBLK = PER_W // P
LG = D // 16


NBUF = 4


def _body(table, idx, out, idx_v, acc, *sems):
    sem_g = sems[0:NBUF]
    sem_o = sems[NBUF : 2 * NBUF]

    cid = lax.axis_index("c")
    sid = lax.axis_index("s")
    wid = sid * NUM_CORES + cid
    obase = wid * PER_W

    pltpu.sync_copy(idx.at[:, pl.ds(wid * PER_W, PER_W)], idx_v)

    def fire_block(f, bb):
        col0 = f * P
        for j in range(C):
            pltpu.async_copy(
                table.at[idx_v.at[j, pl.ds(col0, P)]],
                acc.at[bb],
                sem_g[bb],
                add=(j > 0),
            )

    def drain_block(f, bb):
        col0 = f * P
        for j in range(C):
            pltpu.make_async_copy(
                table.at[idx_v.at[j, pl.ds(col0, P)]],
                acc.at[bb],
                sem_g[bb],
            ).wait()

    def fire_out(f, bb):
        pltpu.async_copy(
            acc.at[bb], out.at[pl.ds(obase + f * P, P)], sem_o[bb]
        )

    def wait_out(f, bb):
        pltpu.make_async_copy(
            acc.at[bb], out.at[pl.ds(obase + f * P, P)], sem_o[bb]
        ).wait()

    for g in range(2):
        fire_block(g, g)

    def body(i, carry):
        for bb in range(NBUF):
            f = NBUF * i + bb - 2
            fg = f + 2
            b_drain = (bb - 2) % NBUF

            @pl.when(jnp.logical_and(f >= 0, f < NBLK))
            def _():
                drain_block(f, b_drain)
                fire_out(f, b_drain)

            @pl.when(jnp.logical_and(fg >= 2, fg < NBLK))
            def _():
                prev = fg - NBUF

                @pl.when(prev >= 0)
                def _():
                    wait_out(prev, bb)

                fire_block(fg, bb)

        return carry

    lax.fori_loop(0, NBLK // NBUF + 1, body, 0)
    for f in range(NBLK - NBUF, NBLK):
        wait_out(f, f % NBUF)


_embed_sum = functools.partial(
    pl.kernel,
    mesh=plsc.VectorSubcoreMesh(
        core_axis_name="c", subcore_axis_name="s",
        num_cores=NUM_CORES, num_subcores=NUM_SUBCORES,
    ),
    out_type=jax.ShapeDtypeStruct((N, D), jnp.float32),
    scratch_types=[
        pltpu.VMEM((C, PER_W), jnp.int32),
        pltpu.VMEM((NBUF, P, D), jnp.float32),
    ]
    + [pltpu.SemaphoreType.DMA] * (2 * NBUF),
    compiler_params=pltpu.CompilerParams(use_tc_tiling_on_sc=False),
)(_body)


@jax.jit
def kernel(node_idx, clause_enc):
    idx_t = node_idx.astype(jnp.int32).reshape(N, C).T
    out = _embed_sum(clause_enc, idx_t)
    return out.reshape(B, L, D)
